# Initial kernel scaffold; baseline (speedup 1.0000x reference)
import jax
import jax.numpy as jnp
from jax.experimental import pallas as pl


def kernel(x, edge_index, edge_type, pos_edge_index, eps, W1, root1, b1, W2, root2, b2, W3, root3, b3, Wd, bd, Wl, bl, Wt, bt):
    raise NotImplementedError("write your pallas kernel here")


# speedup vs baseline: 6.9337x; 6.9337x over previous
import functools

import jax
import jax.numpy as jnp
from jax import lax
from jax.experimental import pallas as pl
from jax.experimental.pallas import tpu as pltpu
from jax.experimental.pallas import tpu_sc as plsc

N = 10000
E = 320000
P = 100000
IN, HID, OUT, R = 128, 256, 128, 8

NUM_SC = 2
NUM_TILES = 16
EPT = E // NUM_TILES
HALF_N = N // NUM_SC
KROWS = 32

_mesh = plsc.VectorSubcoreMesh(core_axis_name="c", subcore_axis_name="s")


SB = 400
QCAP = SB + KROWS
NCOLS = 272
NC = 500
NCHUNK = HALF_N // NC


def _seg_agg_body(tab_h, esrc_h, edst_h, etyp_h, zb_h,
                  out_h, sbuf, dbuf, tbuf, qsrc, qcid, rows, zb, acc, gsem):
    c = lax.axis_index("c")
    s = lax.axis_index("s")
    rpt = (R * NC) // NUM_TILES
    ebase = s * EPT
    pltpu.sync_copy(zb_h, zb)
    node0 = c * HALF_N
    lane = jnp.arange(16, dtype=jnp.int32)
    pad_cid = R * NC + lane
    nz = (rpt + 15) // 16

    def chunk_body(chunk, _c):
        lo = node0 + chunk * NC
        zbase = s * rpt
        for zi in range(nz):
            nrow = min(16, rpt - zi * 16)
            pltpu.sync_copy(zb.at[pl.ds(0, nrow)],
                            acc.at[pl.ds(zbase + zi * 16, nrow)])
        plsc.subcore_barrier()

        def block_body(b, _b):
            off = ebase + b * SB
            pltpu.sync_copy(esrc_h.at[pl.ds(off, SB)], sbuf)
            pltpu.sync_copy(edst_h.at[pl.ds(off, SB)], dbuf)
            pltpu.sync_copy(etyp_h.at[pl.ds(off, SB)], tbuf)

            def scan_body(i, qn):
                sv = sbuf[pl.ds(i * 16, 16)]
                dv = dbuf[pl.ds(i * 16, 16)]
                tv = tbuf[pl.ds(i * 16, 16)]
                m = (dv >= lo) & (dv < lo + NC)
                mi = m.astype(jnp.int32)
                pos = qn + plsc.cumsum(mi) - 1
                gid = tv * N + sv
                cid = tv * NC + (dv - lo)
                plsc.store_scatter(qsrc, [pos], gid, mask=m)
                plsc.store_scatter(qcid, [pos], cid, mask=m)
                return qn + jnp.sum(mi)

            qn = lax.fori_loop(0, SB // 16, scan_body, jnp.int32(0))

            for t in range(KROWS // 16):
                qsrc[pl.ds(qn + t * 16, 16)] = lane
                qcid[pl.ds(qn + t * 16, 16)] = pad_cid

            def drain(j, _):
                cp = pltpu.async_copy(
                    tab_h.at[qsrc.at[pl.ds(j * KROWS, KROWS)]], rows, gsem)
                cp.wait()
                for kk in range(KROWS // 16):
                    cvec = qcid[pl.ds(j * KROWS + kk * 16, 16)]
                    pltpu.sync_copy(rows.at[pl.ds(kk * 16, 16)], acc.at[cvec],
                                    add=True)
                return 0

            nq = (qn + KROWS - 1) // KROWS
            lax.fori_loop(0, nq, drain, 0)
            return 0

        lax.fori_loop(0, EPT // SB, block_body, 0)
        plsc.subcore_barrier()

        rr = (s * rpt) // NC
        roff = (s * rpt) % NC
        pltpu.sync_copy(acc.at[pl.ds(s * rpt, rpt)],
                        out_h.at[rr, pl.ds(lo + roff, rpt)])
        return 0

    lax.fori_loop(0, NCHUNK, chunk_body, 0)


_seg_agg = pl.kernel(
    _seg_agg_body,
    out_type=jax.ShapeDtypeStruct((R, N, NCOLS), jnp.float32),
    mesh=_mesh,
    scratch_types=[
        pltpu.VMEM((SB,), jnp.int32),
        pltpu.VMEM((SB,), jnp.int32),
        pltpu.VMEM((SB,), jnp.int32),
        pltpu.VMEM((QCAP,), jnp.int32),
        pltpu.VMEM((QCAP,), jnp.int32),
        pltpu.VMEM((KROWS, NCOLS), jnp.float32),
        pltpu.VMEM((16, NCOLS), jnp.float32),
        pltpu.VMEM_SHARED((R * NC + 16, NCOLS), jnp.float32),
        pltpu.SemaphoreType.DMA,
    ],
    compiler_params=pltpu.CompilerParams(
        use_tc_tiling_on_sc=False, needs_layout_passes=False),
)


PPAD = 100352
PPW = PPAD // (NUM_SC * NUM_TILES)
KP = 32


def _pair_gather_body(psrc_h, pdst_h, z_h, zs_h, zd_h,
                      sidx, didx, zsb, zdb, sem1, sem2):
    c = lax.axis_index("c")
    s = lax.axis_index("s")
    wid = s * NUM_SC + c
    base = wid * PPW
    pltpu.sync_copy(psrc_h.at[pl.ds(base, PPW)], sidx)
    pltpu.sync_copy(pdst_h.at[pl.ds(base, PPW)], didx)

    def loop(j, _):
        g1 = pltpu.async_copy(z_h.at[sidx.at[pl.ds(j * KP, KP)]], zsb, sem1)
        g2 = pltpu.async_copy(z_h.at[didx.at[pl.ds(j * KP, KP)]], zdb, sem2)
        g1.wait()
        g2.wait()
        pltpu.sync_copy(zsb, zs_h.at[pl.ds(base + j * KP, KP)])
        pltpu.sync_copy(zdb, zd_h.at[pl.ds(base + j * KP, KP)])
        return 0

    lax.fori_loop(0, PPW // KP, loop, 0)


_pair_gather = pl.kernel(
    _pair_gather_body,
    out_type=(jax.ShapeDtypeStruct((PPAD, OUT), jnp.float32),
              jax.ShapeDtypeStruct((PPAD, OUT), jnp.float32)),
    mesh=_mesh,
    scratch_types=[
        pltpu.VMEM((PPW,), jnp.int32),
        pltpu.VMEM((PPW,), jnp.int32),
        pltpu.VMEM((KP, OUT), jnp.float32),
        pltpu.VMEM((KP, OUT), jnp.float32),
        pltpu.SemaphoreType.DMA,
        pltpu.SemaphoreType.DMA,
    ],
    compiler_params=pltpu.CompilerParams(
        use_tc_tiling_on_sc=False, needs_layout_passes=False),
)


BN = 400


def _table1_body(x_ref, w_ref, out_ref):
    t = jnp.dot(x_ref[...], w_ref[0], preferred_element_type=jnp.float32)
    col = lax.broadcasted_iota(jnp.int32, (BN, NCOLS), 1)
    out_ref[0] = jnp.where(col < HID, jnp.pad(t, ((0, 0), (0, 16))),
                           jnp.where(col == HID, 1.0, 0.0))


def _table23_body(h_ref, w2_ref, w3_ref, out_ref):
    t2 = jnp.dot(h_ref[...], w2_ref[0], preferred_element_type=jnp.float32)
    t3 = jnp.dot(h_ref[...], w3_ref[0], preferred_element_type=jnp.float32)
    t = jnp.concatenate([t2, t3], axis=1)
    col = lax.broadcasted_iota(jnp.int32, (BN, NCOLS), 1)
    out_ref[0] = jnp.where(col < 2 * OUT, jnp.pad(t, ((0, 0), (0, 16))),
                           jnp.where(col == 2 * OUT, 1.0, 0.0))


def _combine1_body(x_ref, agg_ref, root_ref, b_ref, out_ref):
    acc = jnp.dot(x_ref[...], root_ref[...],
                  preferred_element_type=jnp.float32)
    acc = acc + b_ref[...]
    for r in range(R):
        a = agg_ref[r]
        cnt = a[:, HID:HID + 1]
        inv = 1.0 / jnp.maximum(cnt, 1.0)
        acc = acc + a[:, :HID] * inv
    out_ref[...] = jnp.maximum(acc, 0.0)


def _combine23_body(h_ref, agg_ref, root2_ref, b2_ref, root3_ref, b3_ref,
                    eps_ref, mean_ref, logstd_ref, z_ref):
    m = jnp.dot(h_ref[...], root2_ref[...], preferred_element_type=jnp.float32)
    m = m + b2_ref[...]
    g = jnp.dot(h_ref[...], root3_ref[...], preferred_element_type=jnp.float32)
    g = g + b3_ref[...]
    for r in range(R):
        a = agg_ref[r]
        cnt = a[:, 2 * OUT:2 * OUT + 1]
        inv = 1.0 / jnp.maximum(cnt, 1.0)
        m = m + a[:, :OUT] * inv
        g = g + a[:, OUT:2 * OUT] * inv
    mean_ref[...] = m
    logstd_ref[...] = g
    z_ref[...] = m + eps_ref[...] * jnp.exp(g)


BP = 512


def _decoder_body(zs_ref, zd_ref, wdt_ref, wdb_ref, bd_ref,
                  wlt_ref, blt_ref, out_ref):
    share = jnp.dot(zs_ref[...], wdt_ref[...],
                    preferred_element_type=jnp.float32)
    share = share + jnp.dot(zd_ref[...], wdb_ref[...],
                            preferred_element_type=jnp.float32)
    share = jnp.maximum(share + bd_ref[...], 0.0)
    o = jnp.dot(share, wlt_ref[...], preferred_element_type=jnp.float32)
    o = o + blt_ref[...]
    col = lax.broadcasted_iota(jnp.int32, (BP, 16), 1)
    out_ref[...] = jnp.where(col == 0, jax.nn.sigmoid(o), o)


def _table1(x, W1):
    return pl.pallas_call(
        _table1_body,
        grid=(R, N // BN),
        in_specs=[
            pl.BlockSpec((BN, IN), lambda r, i: (i, 0)),
            pl.BlockSpec((1, IN, HID), lambda r, i: (r, 0, 0)),
        ],
        out_specs=pl.BlockSpec((1, BN, NCOLS), lambda r, i: (r, i, 0)),
        out_shape=jax.ShapeDtypeStruct((R, N, NCOLS), jnp.float32),
    )(x, W1)


def _table23(h, W2, W3):
    return pl.pallas_call(
        _table23_body,
        grid=(R, N // BN),
        in_specs=[
            pl.BlockSpec((BN, HID), lambda r, i: (i, 0)),
            pl.BlockSpec((1, HID, OUT), lambda r, i: (r, 0, 0)),
            pl.BlockSpec((1, HID, OUT), lambda r, i: (r, 0, 0)),
        ],
        out_specs=pl.BlockSpec((1, BN, NCOLS), lambda r, i: (r, i, 0)),
        out_shape=jax.ShapeDtypeStruct((R, N, NCOLS), jnp.float32),
    )(h, W2, W3)


def _combine1(x, agg1, root1, b1):
    return pl.pallas_call(
        _combine1_body,
        grid=(N // BN,),
        in_specs=[
            pl.BlockSpec((BN, IN), lambda i: (i, 0)),
            pl.BlockSpec((R, BN, NCOLS), lambda i: (0, i, 0)),
            pl.BlockSpec((IN, HID), lambda i: (0, 0)),
            pl.BlockSpec((1, HID), lambda i: (0, 0)),
        ],
        out_specs=pl.BlockSpec((BN, HID), lambda i: (i, 0)),
        out_shape=jax.ShapeDtypeStruct((N, HID), jnp.float32),
    )(x, agg1, root1, b1)


def _combine23(h, agg2, root2, b2, root3, b3, eps):
    out128 = jax.ShapeDtypeStruct((N, OUT), jnp.float32)
    spec128 = pl.BlockSpec((BN, OUT), lambda i: (i, 0))
    return pl.pallas_call(
        _combine23_body,
        grid=(N // BN,),
        in_specs=[
            pl.BlockSpec((BN, HID), lambda i: (i, 0)),
            pl.BlockSpec((R, BN, NCOLS), lambda i: (0, i, 0)),
            pl.BlockSpec((HID, OUT), lambda i: (0, 0)),
            pl.BlockSpec((1, OUT), lambda i: (0, 0)),
            pl.BlockSpec((HID, OUT), lambda i: (0, 0)),
            pl.BlockSpec((1, OUT), lambda i: (0, 0)),
            pl.BlockSpec((BN, OUT), lambda i: (i, 0)),
        ],
        out_specs=(spec128, spec128, spec128),
        out_shape=(out128, out128, out128),
    )(h, agg2, root2, b2, root3, b3, eps)


def _decoder(zs, zd, Wdt, Wdb, bd, Wlt, blt):
    return pl.pallas_call(
        _decoder_body,
        grid=(PPAD // BP,),
        in_specs=[
            pl.BlockSpec((BP, OUT), lambda i: (i, 0)),
            pl.BlockSpec((BP, OUT), lambda i: (i, 0)),
            pl.BlockSpec((OUT, 128), lambda i: (0, 0)),
            pl.BlockSpec((OUT, 128), lambda i: (0, 0)),
            pl.BlockSpec((1, 128), lambda i: (0, 0)),
            pl.BlockSpec((128, 16), lambda i: (0, 0)),
            pl.BlockSpec((1, 16), lambda i: (0, 0)),
        ],
        out_specs=pl.BlockSpec((BP, 16), lambda i: (i, 0)),
        out_shape=jax.ShapeDtypeStruct((PPAD, 16), jnp.float32),
    )(zs, zd, Wdt, Wdb, bd, Wlt, blt)


def kernel(x, edge_index, edge_type, pos_edge_index, eps,
           W1, root1, b1, W2, root2, b2, W3, root3, b3,
           Wd, bd, Wl, bl, Wt, bt):
    esrc = edge_index[0]
    edst = edge_index[1]
    etyp = edge_type
    zb = jnp.zeros((16, NCOLS), jnp.float32)

    t1 = _table1(x, W1).reshape(R * N, NCOLS)
    agg1 = _seg_agg(t1, esrc, edst, etyp, zb)
    h = _combine1(x, agg1, root1, b1.reshape(1, HID))
    t23 = _table23(h, W2, W3).reshape(R * N, NCOLS)
    agg23 = _seg_agg(t23, esrc, edst, etyp, zb)
    mean, logstd, z = _combine23(h, agg23, root2, b2.reshape(1, OUT),
                                 root3, b3.reshape(1, OUT), eps)

    npad = PPAD - P
    psrc = jnp.pad(pos_edge_index[0], (0, npad))
    pdst = jnp.pad(pos_edge_index[1], (0, npad))
    zs, zd = _pair_gather(psrc, pdst, z)

    Wlt = jnp.concatenate([Wl, Wt, jnp.zeros((128, 7), jnp.float32)], axis=1)
    blt = jnp.concatenate([bl, bt, jnp.zeros((7,), jnp.float32)]).reshape(1, 16)
    out16 = _decoder(zs, zd, Wd[:OUT], Wd[OUT:], bd.reshape(1, 128), Wlt, blt)

    link_pred = out16[:P, 0]
    type_pred = out16[:P, 1:1 + R]
    return (link_pred, type_pred, mean, logstd, z)


# speedup vs baseline: 11.9948x; 1.7299x over previous
import functools

import jax
import jax.numpy as jnp
from jax import lax
from jax.experimental import pallas as pl
from jax.experimental.pallas import tpu as pltpu
from jax.experimental.pallas import tpu_sc as plsc

N = 10000
E = 320000
P = 100000
IN, HID, OUT, R = 128, 256, 128, 8

NUM_SC = 2
NUM_TILES = 16
EPT = E // NUM_TILES
HALF_N = N // NUM_SC
KROWS = 32

_mesh = plsc.VectorSubcoreMesh(core_axis_name="c", subcore_axis_name="s")


SB = 2000
NBUF = 4
QPAD = NBUF * KROWS
QCAP = SB + QPAD
NCOLS = 272
NC = 500
NCHUNK = HALF_N // NC


def _seg_agg_body(tab_h, esrc_h, edst_h, etyp_h, zb_h,
                  out_h, sbuf, dbuf, tbuf, qsrc, qcid, rows, zb, acc, gsem):
    c = lax.axis_index("c")
    s = lax.axis_index("s")
    rpt = (R * NC) // NUM_TILES
    ebase = s * EPT
    pltpu.sync_copy(zb_h, zb)
    node0 = c * HALF_N
    lane = jnp.arange(16, dtype=jnp.int32)
    pad_cid = R * NC + lane
    nz = (rpt + 15) // 16

    def chunk_body(chunk, _c):
        lo = node0 + chunk * NC
        zbase = s * rpt
        for zi in range(nz):
            nrow = min(16, rpt - zi * 16)
            pltpu.sync_copy(zb.at[pl.ds(0, nrow)],
                            acc.at[pl.ds(zbase + zi * 16, nrow)])
        plsc.subcore_barrier()

        def block_body(b, _b):
            off = ebase + b * SB
            pltpu.sync_copy(esrc_h.at[pl.ds(off, SB)], sbuf)
            pltpu.sync_copy(edst_h.at[pl.ds(off, SB)], dbuf)
            pltpu.sync_copy(etyp_h.at[pl.ds(off, SB)], tbuf)

            def scan_body(i, qn):
                sv = sbuf[pl.ds(i * 16, 16)]
                dv = dbuf[pl.ds(i * 16, 16)]
                tv = tbuf[pl.ds(i * 16, 16)]
                m = (dv >= lo) & (dv < lo + NC)
                mi = m.astype(jnp.int32)
                pos = qn + plsc.cumsum(mi) - 1
                gid = tv * N + sv
                cid = tv * NC + (dv - lo)
                plsc.store_scatter(qsrc, [pos], gid, mask=m)
                plsc.store_scatter(qcid, [pos], cid, mask=m)
                return qn + jnp.sum(mi)

            qn = lax.fori_loop(0, SB // 16, scan_body, jnp.int32(0))

            for t in range(QPAD // 16):
                qsrc[pl.ds(qn + t * 16, 16)] = lane + 16 * t
                qcid[pl.ds(qn + t * 16, 16)] = pad_cid

            def drain(j4, _):
                qoff = j4 * (NBUF * KROWS)
                cps = [
                    pltpu.async_copy(
                        tab_h.at[qsrc.at[pl.ds(qoff + t * KROWS, KROWS)]],
                        rows.at[t], gsem)
                    for t in range(NBUF)
                ]
                for t in range(NBUF):
                    cps[t].wait()
                    for kk in range(KROWS // 16):
                        cvec = qcid[pl.ds(qoff + t * KROWS + kk * 16, 16)]
                        pltpu.sync_copy(rows.at[t, pl.ds(kk * 16, 16)],
                                        acc.at[cvec], add=True)
                return 0

            nq4 = (qn + QPAD - 1) // QPAD
            lax.fori_loop(0, nq4, drain, 0)
            return 0

        lax.fori_loop(0, EPT // SB, block_body, 0)
        plsc.subcore_barrier()

        rr = (s * rpt) // NC
        roff = (s * rpt) % NC
        pltpu.sync_copy(acc.at[pl.ds(s * rpt, rpt)],
                        out_h.at[rr, pl.ds(lo + roff, rpt)])
        return 0

    lax.fori_loop(0, NCHUNK, chunk_body, 0)


_seg_agg = pl.kernel(
    _seg_agg_body,
    out_type=jax.ShapeDtypeStruct((R, N, NCOLS), jnp.float32),
    mesh=_mesh,
    scratch_types=[
        pltpu.VMEM((SB,), jnp.int32),
        pltpu.VMEM((SB,), jnp.int32),
        pltpu.VMEM((SB,), jnp.int32),
        pltpu.VMEM((QCAP,), jnp.int32),
        pltpu.VMEM((QCAP,), jnp.int32),
        pltpu.VMEM((NBUF, KROWS, NCOLS), jnp.float32),
        pltpu.VMEM((16, NCOLS), jnp.float32),
        pltpu.VMEM_SHARED((R * NC + 16, NCOLS), jnp.float32),
        pltpu.SemaphoreType.DMA,
    ],
    compiler_params=pltpu.CompilerParams(
        use_tc_tiling_on_sc=False, needs_layout_passes=False),
)


PPAD = 100352
PPW = PPAD // (NUM_SC * NUM_TILES)
KP = 32


def _pair_gather_body(psrc_h, pdst_h, z_h, zs_h, zd_h,
                      sidx, didx, zsb, zdb, sem1, sem2):
    c = lax.axis_index("c")
    s = lax.axis_index("s")
    wid = s * NUM_SC + c
    base = wid * PPW
    pltpu.sync_copy(psrc_h.at[pl.ds(base, PPW)], sidx)
    pltpu.sync_copy(pdst_h.at[pl.ds(base, PPW)], didx)

    def loop(j, _):
        cps = []
        for t in range(2):
            q = j * 2 * KP + t * KP
            cps.append((
                pltpu.async_copy(z_h.at[sidx.at[pl.ds(q, KP)]],
                                 zsb.at[t], sem1),
                pltpu.async_copy(z_h.at[didx.at[pl.ds(q, KP)]],
                                 zdb.at[t], sem2),
            ))
        for t in range(2):
            g1, g2 = cps[t]
            g1.wait()
            g2.wait()
            q = base + j * 2 * KP + t * KP
            pltpu.sync_copy(zsb.at[t], zs_h.at[pl.ds(q, KP)])
            pltpu.sync_copy(zdb.at[t], zd_h.at[pl.ds(q, KP)])
        return 0

    lax.fori_loop(0, PPW // (2 * KP), loop, 0)


_pair_gather = pl.kernel(
    _pair_gather_body,
    out_type=(jax.ShapeDtypeStruct((PPAD, OUT), jnp.float32),
              jax.ShapeDtypeStruct((PPAD, OUT), jnp.float32)),
    mesh=_mesh,
    scratch_types=[
        pltpu.VMEM((PPW,), jnp.int32),
        pltpu.VMEM((PPW,), jnp.int32),
        pltpu.VMEM((2, KP, OUT), jnp.float32),
        pltpu.VMEM((2, KP, OUT), jnp.float32),
        pltpu.SemaphoreType.DMA,
        pltpu.SemaphoreType.DMA,
    ],
    compiler_params=pltpu.CompilerParams(
        use_tc_tiling_on_sc=False, needs_layout_passes=False),
)


BN = 400


def _table1_body(x_ref, w_ref, out_ref):
    t = jnp.dot(x_ref[...], w_ref[0], preferred_element_type=jnp.float32)
    col = lax.broadcasted_iota(jnp.int32, (BN, NCOLS), 1)
    out_ref[0] = jnp.where(col < HID, jnp.pad(t, ((0, 0), (0, 16))),
                           jnp.where(col == HID, 1.0, 0.0))


def _table23_body(h_ref, w2_ref, w3_ref, out_ref):
    t2 = jnp.dot(h_ref[...], w2_ref[0], preferred_element_type=jnp.float32)
    t3 = jnp.dot(h_ref[...], w3_ref[0], preferred_element_type=jnp.float32)
    t = jnp.concatenate([t2, t3], axis=1)
    col = lax.broadcasted_iota(jnp.int32, (BN, NCOLS), 1)
    out_ref[0] = jnp.where(col < 2 * OUT, jnp.pad(t, ((0, 0), (0, 16))),
                           jnp.where(col == 2 * OUT, 1.0, 0.0))


def _combine1_body(x_ref, agg_ref, root_ref, b_ref, out_ref):
    acc = jnp.dot(x_ref[...], root_ref[...],
                  preferred_element_type=jnp.float32)
    acc = acc + b_ref[...]
    for r in range(R):
        a = agg_ref[r]
        cnt = a[:, HID:HID + 1]
        inv = 1.0 / jnp.maximum(cnt, 1.0)
        acc = acc + a[:, :HID] * inv
    out_ref[...] = jnp.maximum(acc, 0.0)


def _combine23_body(h_ref, agg_ref, root2_ref, b2_ref, root3_ref, b3_ref,
                    eps_ref, mean_ref, logstd_ref, z_ref):
    m = jnp.dot(h_ref[...], root2_ref[...], preferred_element_type=jnp.float32)
    m = m + b2_ref[...]
    g = jnp.dot(h_ref[...], root3_ref[...], preferred_element_type=jnp.float32)
    g = g + b3_ref[...]
    for r in range(R):
        a = agg_ref[r]
        cnt = a[:, 2 * OUT:2 * OUT + 1]
        inv = 1.0 / jnp.maximum(cnt, 1.0)
        m = m + a[:, :OUT] * inv
        g = g + a[:, OUT:2 * OUT] * inv
    mean_ref[...] = m
    logstd_ref[...] = g
    z_ref[...] = m + eps_ref[...] * jnp.exp(g)


BP = 512


def _decoder_body(zs_ref, zd_ref, wdt_ref, wdb_ref, bd_ref,
                  wlt_ref, blt_ref, out_ref):
    share = jnp.dot(zs_ref[...], wdt_ref[...],
                    preferred_element_type=jnp.float32)
    share = share + jnp.dot(zd_ref[...], wdb_ref[...],
                            preferred_element_type=jnp.float32)
    share = jnp.maximum(share + bd_ref[...], 0.0)
    o = jnp.dot(share, wlt_ref[...], preferred_element_type=jnp.float32)
    o = o + blt_ref[...]
    col = lax.broadcasted_iota(jnp.int32, (BP, 16), 1)
    out_ref[...] = jnp.where(col == 0, jax.nn.sigmoid(o), o)


def _table1(x, W1):
    return pl.pallas_call(
        _table1_body,
        grid=(R, N // BN),
        in_specs=[
            pl.BlockSpec((BN, IN), lambda r, i: (i, 0)),
            pl.BlockSpec((1, IN, HID), lambda r, i: (r, 0, 0)),
        ],
        out_specs=pl.BlockSpec((1, BN, NCOLS), lambda r, i: (r, i, 0)),
        out_shape=jax.ShapeDtypeStruct((R, N, NCOLS), jnp.float32),
    )(x, W1)


def _table23(h, W2, W3):
    return pl.pallas_call(
        _table23_body,
        grid=(R, N // BN),
        in_specs=[
            pl.BlockSpec((BN, HID), lambda r, i: (i, 0)),
            pl.BlockSpec((1, HID, OUT), lambda r, i: (r, 0, 0)),
            pl.BlockSpec((1, HID, OUT), lambda r, i: (r, 0, 0)),
        ],
        out_specs=pl.BlockSpec((1, BN, NCOLS), lambda r, i: (r, i, 0)),
        out_shape=jax.ShapeDtypeStruct((R, N, NCOLS), jnp.float32),
    )(h, W2, W3)


def _combine1(x, agg1, root1, b1):
    return pl.pallas_call(
        _combine1_body,
        grid=(N // BN,),
        in_specs=[
            pl.BlockSpec((BN, IN), lambda i: (i, 0)),
            pl.BlockSpec((R, BN, NCOLS), lambda i: (0, i, 0)),
            pl.BlockSpec((IN, HID), lambda i: (0, 0)),
            pl.BlockSpec((1, HID), lambda i: (0, 0)),
        ],
        out_specs=pl.BlockSpec((BN, HID), lambda i: (i, 0)),
        out_shape=jax.ShapeDtypeStruct((N, HID), jnp.float32),
    )(x, agg1, root1, b1)


def _combine23(h, agg2, root2, b2, root3, b3, eps):
    out128 = jax.ShapeDtypeStruct((N, OUT), jnp.float32)
    spec128 = pl.BlockSpec((BN, OUT), lambda i: (i, 0))
    return pl.pallas_call(
        _combine23_body,
        grid=(N // BN,),
        in_specs=[
            pl.BlockSpec((BN, HID), lambda i: (i, 0)),
            pl.BlockSpec((R, BN, NCOLS), lambda i: (0, i, 0)),
            pl.BlockSpec((HID, OUT), lambda i: (0, 0)),
            pl.BlockSpec((1, OUT), lambda i: (0, 0)),
            pl.BlockSpec((HID, OUT), lambda i: (0, 0)),
            pl.BlockSpec((1, OUT), lambda i: (0, 0)),
            pl.BlockSpec((BN, OUT), lambda i: (i, 0)),
        ],
        out_specs=(spec128, spec128, spec128),
        out_shape=(out128, out128, out128),
    )(h, agg2, root2, b2, root3, b3, eps)


def _decoder(zs, zd, Wdt, Wdb, bd, Wlt, blt):
    return pl.pallas_call(
        _decoder_body,
        grid=(PPAD // BP,),
        in_specs=[
            pl.BlockSpec((BP, OUT), lambda i: (i, 0)),
            pl.BlockSpec((BP, OUT), lambda i: (i, 0)),
            pl.BlockSpec((OUT, 128), lambda i: (0, 0)),
            pl.BlockSpec((OUT, 128), lambda i: (0, 0)),
            pl.BlockSpec((1, 128), lambda i: (0, 0)),
            pl.BlockSpec((128, 16), lambda i: (0, 0)),
            pl.BlockSpec((1, 16), lambda i: (0, 0)),
        ],
        out_specs=pl.BlockSpec((BP, 16), lambda i: (i, 0)),
        out_shape=jax.ShapeDtypeStruct((PPAD, 16), jnp.float32),
    )(zs, zd, Wdt, Wdb, bd, Wlt, blt)


def kernel(x, edge_index, edge_type, pos_edge_index, eps,
           W1, root1, b1, W2, root2, b2, W3, root3, b3,
           Wd, bd, Wl, bl, Wt, bt):
    esrc = edge_index[0]
    edst = edge_index[1]
    etyp = edge_type
    zb = jnp.zeros((16, NCOLS), jnp.float32)

    t1 = _table1(x, W1).reshape(R * N, NCOLS)
    agg1 = _seg_agg(t1, esrc, edst, etyp, zb)
    h = _combine1(x, agg1, root1, b1.reshape(1, HID))
    t23 = _table23(h, W2, W3).reshape(R * N, NCOLS)
    agg23 = _seg_agg(t23, esrc, edst, etyp, zb)
    mean, logstd, z = _combine23(h, agg23, root2, b2.reshape(1, OUT),
                                 root3, b3.reshape(1, OUT), eps)

    npad = PPAD - P
    psrc = jnp.pad(pos_edge_index[0], (0, npad))
    pdst = jnp.pad(pos_edge_index[1], (0, npad))
    zs, zd = _pair_gather(psrc, pdst, z)

    Wlt = jnp.concatenate([Wl, Wt, jnp.zeros((128, 7), jnp.float32)], axis=1)
    blt = jnp.concatenate([bl, bt, jnp.zeros((7,), jnp.float32)]).reshape(1, 16)
    out16 = _decoder(zs, zd, Wd[:OUT], Wd[OUT:], bd.reshape(1, 128), Wlt, blt)

    link_pred = out16[:P, 0]
    type_pred = out16[:P, 1:1 + R]
    return (link_pred, type_pred, mean, logstd, z)


# speedup vs baseline: 13.6437x; 1.1375x over previous
import functools

import jax
import jax.numpy as jnp
from jax import lax
from jax.experimental import pallas as pl
from jax.experimental.pallas import tpu as pltpu
from jax.experimental.pallas import tpu_sc as plsc

N = 10000
E = 320000
P = 100000
IN, HID, OUT, R = 128, 256, 128, 8

NUM_SC = 2
NUM_TILES = 16
EPT = E // NUM_TILES
HALF_N = N // NUM_SC
KROWS = 32

_mesh = plsc.VectorSubcoreMesh(core_axis_name="c", subcore_axis_name="s")


SB = 2000
NBUF = 4
QPAD = NBUF * KROWS
QCAP = SB + QPAD
NCOLS = 272
NC = 500
NCHUNK = HALF_N // NC


def _seg_agg_body(tab_h, esrc_h, edst_h, etyp_h, zb_h,
                  out_h, sbuf, dbuf, tbuf, qsrc, qcid, rows, zb, acc, gsem,
                  ssem):
    c = lax.axis_index("c")
    s = lax.axis_index("s")
    rpt = (R * NC) // NUM_TILES
    ebase = s * EPT
    pltpu.sync_copy(zb_h, zb)
    node0 = c * HALF_N
    lane = jnp.arange(16, dtype=jnp.int32)
    pad_cid = R * NC + lane
    nz = (rpt + 15) // 16

    def chunk_body(chunk, _c):
        lo = node0 + chunk * NC
        zbase = s * rpt
        for zi in range(nz):
            nrow = min(16, rpt - zi * 16)
            pltpu.sync_copy(zb.at[pl.ds(0, nrow)],
                            acc.at[pl.ds(zbase + zi * 16, nrow)])
        plsc.subcore_barrier()

        for eh, eb in ((esrc_h, sbuf), (edst_h, dbuf), (etyp_h, tbuf)):
            pltpu.async_copy(eh.at[pl.ds(ebase, SB)], eb.at[0], ssem)

        def block_body(b, _b):
            cur = lax.rem(b, 2)
            off = ebase + b * SB
            for eh, eb in ((esrc_h, sbuf), (edst_h, dbuf), (etyp_h, tbuf)):
                pltpu.make_async_copy(eh.at[pl.ds(off, SB)], eb.at[cur],
                                      ssem).wait()

            @pl.when(b + 1 < EPT // SB)
            def _prefetch():
                noff = off + SB
                nxt = lax.rem(b + 1, 2)
                for eh, eb in ((esrc_h, sbuf), (edst_h, dbuf), (etyp_h, tbuf)):
                    pltpu.async_copy(eh.at[pl.ds(noff, SB)], eb.at[nxt], ssem)

            def scan_body(i, qn):
                for u in range(5):
                    q = i * 80 + u * 16
                    sv = sbuf[cur, pl.ds(q, 16)]
                    dv = dbuf[cur, pl.ds(q, 16)]
                    tv = tbuf[cur, pl.ds(q, 16)]
                    m = (dv >= lo) & (dv < lo + NC)
                    mi = m.astype(jnp.int32)
                    pos = qn + plsc.cumsum(mi) - 1
                    gid = tv * N + sv
                    cid = tv * NC + (dv - lo)
                    plsc.store_scatter(qsrc, [pos], gid, mask=m)
                    plsc.store_scatter(qcid, [pos], cid, mask=m)
                    qn = qn + jnp.sum(mi)
                return qn

            qn = lax.fori_loop(0, SB // 80, scan_body, jnp.int32(0))

            for t in range(QPAD // 16):
                qsrc[pl.ds(qn + t * 16, 16)] = lane + 16 * t
                qcid[pl.ds(qn + t * 16, 16)] = pad_cid

            def drain(j4, _):
                qoff = j4 * (NBUF * KROWS)
                cps = [
                    pltpu.async_copy(
                        tab_h.at[qsrc.at[pl.ds(qoff + t * KROWS, KROWS)]],
                        rows.at[t], gsem)
                    for t in range(NBUF)
                ]
                for t in range(NBUF):
                    cps[t].wait()
                    for kk in range(KROWS // 16):
                        cvec = qcid[pl.ds(qoff + t * KROWS + kk * 16, 16)]
                        pltpu.sync_copy(rows.at[t, pl.ds(kk * 16, 16)],
                                        acc.at[cvec], add=True)
                return 0

            nq4 = (qn + QPAD - 1) // QPAD
            lax.fori_loop(0, nq4, drain, 0)
            return 0

        lax.fori_loop(0, EPT // SB, block_body, 0)
        plsc.subcore_barrier()

        rr = (s * rpt) // NC
        roff = (s * rpt) % NC
        pltpu.sync_copy(acc.at[pl.ds(s * rpt, rpt)],
                        out_h.at[rr, pl.ds(lo + roff, rpt)])
        return 0

    lax.fori_loop(0, NCHUNK, chunk_body, 0)


_seg_agg = pl.kernel(
    _seg_agg_body,
    out_type=jax.ShapeDtypeStruct((R, N, NCOLS), jnp.float32),
    mesh=_mesh,
    scratch_types=[
        pltpu.VMEM((2, SB), jnp.int32),
        pltpu.VMEM((2, SB), jnp.int32),
        pltpu.VMEM((2, SB), jnp.int32),
        pltpu.VMEM((QCAP,), jnp.int32),
        pltpu.VMEM((QCAP,), jnp.int32),
        pltpu.VMEM((NBUF, KROWS, NCOLS), jnp.float32),
        pltpu.VMEM((16, NCOLS), jnp.float32),
        pltpu.VMEM_SHARED((R * NC + 16, NCOLS), jnp.float32),
        pltpu.SemaphoreType.DMA,
        pltpu.SemaphoreType.DMA,
    ],
    compiler_params=pltpu.CompilerParams(
        use_tc_tiling_on_sc=False, needs_layout_passes=False),
)


PPAD = 100352
PPW = PPAD // (NUM_SC * NUM_TILES)
KP = 32


def _pair_gather_body(psrc_h, pdst_h, z_h, zs_h, zd_h,
                      sidx, didx, zsb, zdb, sem1, sem2):
    c = lax.axis_index("c")
    s = lax.axis_index("s")
    wid = s * NUM_SC + c
    base = wid * PPW
    pltpu.sync_copy(psrc_h.at[pl.ds(base, PPW)], sidx)
    pltpu.sync_copy(pdst_h.at[pl.ds(base, PPW)], didx)

    def loop(j, _):
        cps = []
        for t in range(2):
            q = j * 2 * KP + t * KP
            cps.append((
                pltpu.async_copy(z_h.at[sidx.at[pl.ds(q, KP)]],
                                 zsb.at[t], sem1),
                pltpu.async_copy(z_h.at[didx.at[pl.ds(q, KP)]],
                                 zdb.at[t], sem2),
            ))
        for t in range(2):
            g1, g2 = cps[t]
            g1.wait()
            g2.wait()
            q = base + j * 2 * KP + t * KP
            pltpu.sync_copy(zsb.at[t], zs_h.at[pl.ds(q, KP)])
            pltpu.sync_copy(zdb.at[t], zd_h.at[pl.ds(q, KP)])
        return 0

    lax.fori_loop(0, PPW // (2 * KP), loop, 0)


_pair_gather = pl.kernel(
    _pair_gather_body,
    out_type=(jax.ShapeDtypeStruct((PPAD, OUT), jnp.float32),
              jax.ShapeDtypeStruct((PPAD, OUT), jnp.float32)),
    mesh=_mesh,
    scratch_types=[
        pltpu.VMEM((PPW,), jnp.int32),
        pltpu.VMEM((PPW,), jnp.int32),
        pltpu.VMEM((2, KP, OUT), jnp.float32),
        pltpu.VMEM((2, KP, OUT), jnp.float32),
        pltpu.SemaphoreType.DMA,
        pltpu.SemaphoreType.DMA,
    ],
    compiler_params=pltpu.CompilerParams(
        use_tc_tiling_on_sc=False, needs_layout_passes=False),
)


BN = 400


def _table1_body(x_ref, w_ref, out_ref):
    t = jnp.dot(x_ref[...], w_ref[0], preferred_element_type=jnp.float32)
    col = lax.broadcasted_iota(jnp.int32, (BN, NCOLS), 1)
    out_ref[0] = jnp.where(col < HID, jnp.pad(t, ((0, 0), (0, 16))),
                           jnp.where(col == HID, 1.0, 0.0))


def _table23_body(h_ref, w2_ref, w3_ref, out_ref):
    t2 = jnp.dot(h_ref[...], w2_ref[0], preferred_element_type=jnp.float32)
    t3 = jnp.dot(h_ref[...], w3_ref[0], preferred_element_type=jnp.float32)
    t = jnp.concatenate([t2, t3], axis=1)
    col = lax.broadcasted_iota(jnp.int32, (BN, NCOLS), 1)
    out_ref[0] = jnp.where(col < 2 * OUT, jnp.pad(t, ((0, 0), (0, 16))),
                           jnp.where(col == 2 * OUT, 1.0, 0.0))


def _combine1_body(x_ref, agg_ref, root_ref, b_ref, out_ref):
    acc = jnp.dot(x_ref[...], root_ref[...],
                  preferred_element_type=jnp.float32)
    acc = acc + b_ref[...]
    for r in range(R):
        a = agg_ref[r]
        cnt = a[:, HID:HID + 1]
        inv = 1.0 / jnp.maximum(cnt, 1.0)
        acc = acc + a[:, :HID] * inv
    out_ref[...] = jnp.maximum(acc, 0.0)


def _combine23_body(h_ref, agg_ref, root2_ref, b2_ref, root3_ref, b3_ref,
                    eps_ref, mean_ref, logstd_ref, z_ref):
    m = jnp.dot(h_ref[...], root2_ref[...], preferred_element_type=jnp.float32)
    m = m + b2_ref[...]
    g = jnp.dot(h_ref[...], root3_ref[...], preferred_element_type=jnp.float32)
    g = g + b3_ref[...]
    for r in range(R):
        a = agg_ref[r]
        cnt = a[:, 2 * OUT:2 * OUT + 1]
        inv = 1.0 / jnp.maximum(cnt, 1.0)
        m = m + a[:, :OUT] * inv
        g = g + a[:, OUT:2 * OUT] * inv
    mean_ref[...] = m
    logstd_ref[...] = g
    z_ref[...] = m + eps_ref[...] * jnp.exp(g)


BP = 512


def _decoder_body(zs_ref, zd_ref, wdt_ref, wdb_ref, bd_ref,
                  wlt_ref, blt_ref, out_ref):
    share = jnp.dot(zs_ref[...], wdt_ref[...],
                    preferred_element_type=jnp.float32)
    share = share + jnp.dot(zd_ref[...], wdb_ref[...],
                            preferred_element_type=jnp.float32)
    share = jnp.maximum(share + bd_ref[...], 0.0)
    o = jnp.dot(share, wlt_ref[...], preferred_element_type=jnp.float32)
    o = o + blt_ref[...]
    col = lax.broadcasted_iota(jnp.int32, (BP, 16), 1)
    out_ref[...] = jnp.where(col == 0, jax.nn.sigmoid(o), o)


def _table1(x, W1):
    return pl.pallas_call(
        _table1_body,
        grid=(R, N // BN),
        in_specs=[
            pl.BlockSpec((BN, IN), lambda r, i: (i, 0)),
            pl.BlockSpec((1, IN, HID), lambda r, i: (r, 0, 0)),
        ],
        out_specs=pl.BlockSpec((1, BN, NCOLS), lambda r, i: (r, i, 0)),
        out_shape=jax.ShapeDtypeStruct((R, N, NCOLS), jnp.float32),
    )(x, W1)


def _table23(h, W2, W3):
    return pl.pallas_call(
        _table23_body,
        grid=(R, N // BN),
        in_specs=[
            pl.BlockSpec((BN, HID), lambda r, i: (i, 0)),
            pl.BlockSpec((1, HID, OUT), lambda r, i: (r, 0, 0)),
            pl.BlockSpec((1, HID, OUT), lambda r, i: (r, 0, 0)),
        ],
        out_specs=pl.BlockSpec((1, BN, NCOLS), lambda r, i: (r, i, 0)),
        out_shape=jax.ShapeDtypeStruct((R, N, NCOLS), jnp.float32),
    )(h, W2, W3)


def _combine1(x, agg1, root1, b1):
    return pl.pallas_call(
        _combine1_body,
        grid=(N // BN,),
        in_specs=[
            pl.BlockSpec((BN, IN), lambda i: (i, 0)),
            pl.BlockSpec((R, BN, NCOLS), lambda i: (0, i, 0)),
            pl.BlockSpec((IN, HID), lambda i: (0, 0)),
            pl.BlockSpec((1, HID), lambda i: (0, 0)),
        ],
        out_specs=pl.BlockSpec((BN, HID), lambda i: (i, 0)),
        out_shape=jax.ShapeDtypeStruct((N, HID), jnp.float32),
    )(x, agg1, root1, b1)


def _combine23(h, agg2, root2, b2, root3, b3, eps):
    out128 = jax.ShapeDtypeStruct((N, OUT), jnp.float32)
    spec128 = pl.BlockSpec((BN, OUT), lambda i: (i, 0))
    return pl.pallas_call(
        _combine23_body,
        grid=(N // BN,),
        in_specs=[
            pl.BlockSpec((BN, HID), lambda i: (i, 0)),
            pl.BlockSpec((R, BN, NCOLS), lambda i: (0, i, 0)),
            pl.BlockSpec((HID, OUT), lambda i: (0, 0)),
            pl.BlockSpec((1, OUT), lambda i: (0, 0)),
            pl.BlockSpec((HID, OUT), lambda i: (0, 0)),
            pl.BlockSpec((1, OUT), lambda i: (0, 0)),
            pl.BlockSpec((BN, OUT), lambda i: (i, 0)),
        ],
        out_specs=(spec128, spec128, spec128),
        out_shape=(out128, out128, out128),
    )(h, agg2, root2, b2, root3, b3, eps)


def _decoder(zs, zd, Wdt, Wdb, bd, Wlt, blt):
    return pl.pallas_call(
        _decoder_body,
        grid=(PPAD // BP,),
        in_specs=[
            pl.BlockSpec((BP, OUT), lambda i: (i, 0)),
            pl.BlockSpec((BP, OUT), lambda i: (i, 0)),
            pl.BlockSpec((OUT, 128), lambda i: (0, 0)),
            pl.BlockSpec((OUT, 128), lambda i: (0, 0)),
            pl.BlockSpec((1, 128), lambda i: (0, 0)),
            pl.BlockSpec((128, 16), lambda i: (0, 0)),
            pl.BlockSpec((1, 16), lambda i: (0, 0)),
        ],
        out_specs=pl.BlockSpec((BP, 16), lambda i: (i, 0)),
        out_shape=jax.ShapeDtypeStruct((PPAD, 16), jnp.float32),
    )(zs, zd, Wdt, Wdb, bd, Wlt, blt)


def kernel(x, edge_index, edge_type, pos_edge_index, eps,
           W1, root1, b1, W2, root2, b2, W3, root3, b3,
           Wd, bd, Wl, bl, Wt, bt):
    esrc = edge_index[0]
    edst = edge_index[1]
    etyp = edge_type
    zb = jnp.zeros((16, NCOLS), jnp.float32)

    t1 = _table1(x, W1).reshape(R * N, NCOLS)
    agg1 = _seg_agg(t1, esrc, edst, etyp, zb)
    h = _combine1(x, agg1, root1, b1.reshape(1, HID))
    t23 = _table23(h, W2, W3).reshape(R * N, NCOLS)
    agg23 = _seg_agg(t23, esrc, edst, etyp, zb)
    mean, logstd, z = _combine23(h, agg23, root2, b2.reshape(1, OUT),
                                 root3, b3.reshape(1, OUT), eps)

    npad = PPAD - P
    psrc = jnp.pad(pos_edge_index[0], (0, npad))
    pdst = jnp.pad(pos_edge_index[1], (0, npad))
    zs, zd = _pair_gather(psrc, pdst, z)

    Wlt = jnp.concatenate([Wl, Wt, jnp.zeros((128, 7), jnp.float32)], axis=1)
    blt = jnp.concatenate([bl, bt, jnp.zeros((7,), jnp.float32)]).reshape(1, 16)
    out16 = _decoder(zs, zd, Wd[:OUT], Wd[OUT:], bd.reshape(1, 128), Wlt, blt)

    link_pred = out16[:P, 0]
    type_pred = out16[:P, 1:1 + R]
    return (link_pred, type_pred, mean, logstd, z)


# speedup vs baseline: 14.4634x; 1.0601x over previous
import functools

import jax
import jax.numpy as jnp
from jax import lax
from jax.experimental import pallas as pl
from jax.experimental.pallas import tpu as pltpu
from jax.experimental.pallas import tpu_sc as plsc

N = 10000
E = 320000
P = 100000
IN, HID, OUT, R = 128, 256, 128, 8

NUM_SC = 2
NUM_TILES = 16
EPT = E // NUM_TILES
HALF_N = N // NUM_SC
KROWS = 32

_mesh = plsc.VectorSubcoreMesh(core_axis_name="c", subcore_axis_name="s")


SB = 2000
NBUF = 4
QPAD = NBUF * KROWS
QCAP = SB + QPAD
NCOLS = 272
NC = 500
NCHUNK = HALF_N // NC


def _seg_agg_body(ncols, tab_h, esrc_h, edst_h, etyp_h, zb_h,
                  out_h, sbuf, dbuf, tbuf, qsrc, qcid, rows, zb, acc, gsem,
                  ssem):
    c = lax.axis_index("c")
    s = lax.axis_index("s")
    rpt = (R * NC) // NUM_TILES
    ebase = s * EPT
    pltpu.sync_copy(zb_h, zb)
    node0 = c * HALF_N
    lane = jnp.arange(16, dtype=jnp.int32)
    pad_cid = R * NC + lane
    nz = (rpt + 15) // 16

    def chunk_body(chunk, _c):
        lo = node0 + chunk * NC
        zbase = s * rpt
        for zi in range(nz):
            nrow = min(16, rpt - zi * 16)
            pltpu.sync_copy(zb.at[pl.ds(0, nrow)],
                            acc.at[pl.ds(zbase + zi * 16, nrow)])
        plsc.subcore_barrier()

        for eh, eb in ((esrc_h, sbuf), (edst_h, dbuf), (etyp_h, tbuf)):
            pltpu.async_copy(eh.at[pl.ds(ebase, SB)], eb.at[0], ssem)

        def block_body(b, _b):
            cur = lax.rem(b, 2)
            off = ebase + b * SB
            for eh, eb in ((esrc_h, sbuf), (edst_h, dbuf), (etyp_h, tbuf)):
                pltpu.make_async_copy(eh.at[pl.ds(off, SB)], eb.at[cur],
                                      ssem).wait()

            @pl.when(b + 1 < EPT // SB)
            def _prefetch():
                noff = off + SB
                nxt = lax.rem(b + 1, 2)
                for eh, eb in ((esrc_h, sbuf), (edst_h, dbuf), (etyp_h, tbuf)):
                    pltpu.async_copy(eh.at[pl.ds(noff, SB)], eb.at[nxt], ssem)

            def scan_body(i, qn):
                for u in range(5):
                    q = i * 80 + u * 16
                    sv = sbuf[cur, pl.ds(q, 16)]
                    dv = dbuf[cur, pl.ds(q, 16)]
                    tv = tbuf[cur, pl.ds(q, 16)]
                    m = (dv >= lo) & (dv < lo + NC)
                    mi = m.astype(jnp.int32)
                    pos = qn + plsc.cumsum(mi) - 1
                    gid = tv * N + sv
                    cid = tv * NC + (dv - lo)
                    plsc.store_scatter(qsrc, [pos], gid, mask=m)
                    plsc.store_scatter(qcid, [pos], cid, mask=m)
                    qn = qn + jnp.sum(mi)
                return qn

            qn = lax.fori_loop(0, SB // 80, scan_body, jnp.int32(0))

            for t in range(QPAD // 16):
                qsrc[pl.ds(qn + t * 16, 16)] = lane + 16 * t
                qcid[pl.ds(qn + t * 16, 16)] = pad_cid

            def drain(j4, _):
                qoff = j4 * (NBUF * KROWS)
                cps = [
                    pltpu.async_copy(
                        tab_h.at[qsrc.at[pl.ds(qoff + t * KROWS, KROWS)]],
                        rows.at[t], gsem)
                    for t in range(NBUF)
                ]
                for t in range(NBUF):
                    cps[t].wait()
                    for kk in range(KROWS // 16):
                        cvec = qcid[pl.ds(qoff + t * KROWS + kk * 16, 16)]
                        pltpu.sync_copy(rows.at[t, pl.ds(kk * 16, 16)],
                                        acc.at[cvec], add=True)
                return 0

            nq4 = (qn + QPAD - 1) // QPAD
            lax.fori_loop(0, nq4, drain, 0)
            return 0

        lax.fori_loop(0, EPT // SB, block_body, 0)
        plsc.subcore_barrier()

        rr = (s * rpt) // NC
        roff = (s * rpt) % NC
        pltpu.sync_copy(acc.at[pl.ds(s * rpt, rpt)],
                        out_h.at[rr, pl.ds(lo + roff, rpt)])
        return 0

    lax.fori_loop(0, NCHUNK, chunk_body, 0)


def _make_seg_agg(ncols):
  return pl.kernel(
    functools.partial(_seg_agg_body, ncols),
    out_type=jax.ShapeDtypeStruct((R, N, ncols), jnp.float32),
    mesh=_mesh,
    scratch_types=[
        pltpu.VMEM((2, SB), jnp.int32),
        pltpu.VMEM((2, SB), jnp.int32),
        pltpu.VMEM((2, SB), jnp.int32),
        pltpu.VMEM((QCAP,), jnp.int32),
        pltpu.VMEM((QCAP,), jnp.int32),
        pltpu.VMEM((NBUF, KROWS, ncols), jnp.float32),
        pltpu.VMEM((16, ncols), jnp.float32),
        pltpu.VMEM_SHARED((R * NC + 16, ncols), jnp.float32),
        pltpu.SemaphoreType.DMA,
        pltpu.SemaphoreType.DMA,
    ],
    compiler_params=pltpu.CompilerParams(
        use_tc_tiling_on_sc=False, needs_layout_passes=False),
  )


_seg_agg1 = _make_seg_agg(NCOLS)
_seg_agg2 = _make_seg_agg(2 * OUT)


PPAD = 100352
PPW = PPAD // (NUM_SC * NUM_TILES)
KP = 32


def _pair_gather_body(psrc_h, pdst_h, z_h, zs_h, zd_h,
                      sidx, didx, zsb, zdb, sem1, sem2):
    c = lax.axis_index("c")
    s = lax.axis_index("s")
    wid = s * NUM_SC + c
    base = wid * PPW
    pltpu.sync_copy(psrc_h.at[pl.ds(base, PPW)], sidx)
    pltpu.sync_copy(pdst_h.at[pl.ds(base, PPW)], didx)

    def loop(j, _):
        cps = []
        for t in range(2):
            q = j * 2 * KP + t * KP
            cps.append((
                pltpu.async_copy(z_h.at[sidx.at[pl.ds(q, KP)]],
                                 zsb.at[t], sem1),
                pltpu.async_copy(z_h.at[didx.at[pl.ds(q, KP)]],
                                 zdb.at[t], sem2),
            ))
        for t in range(2):
            g1, g2 = cps[t]
            g1.wait()
            g2.wait()
            q = base + j * 2 * KP + t * KP
            pltpu.sync_copy(zsb.at[t], zs_h.at[pl.ds(q, KP)])
            pltpu.sync_copy(zdb.at[t], zd_h.at[pl.ds(q, KP)])
        return 0

    lax.fori_loop(0, PPW // (2 * KP), loop, 0)


_pair_gather = pl.kernel(
    _pair_gather_body,
    out_type=(jax.ShapeDtypeStruct((PPAD, OUT), jnp.float32),
              jax.ShapeDtypeStruct((PPAD, OUT), jnp.float32)),
    mesh=_mesh,
    scratch_types=[
        pltpu.VMEM((PPW,), jnp.int32),
        pltpu.VMEM((PPW,), jnp.int32),
        pltpu.VMEM((2, KP, OUT), jnp.float32),
        pltpu.VMEM((2, KP, OUT), jnp.float32),
        pltpu.SemaphoreType.DMA,
        pltpu.SemaphoreType.DMA,
    ],
    compiler_params=pltpu.CompilerParams(
        use_tc_tiling_on_sc=False, needs_layout_passes=False),
)


BN = 400


def _table1_body(x_ref, w_ref, out_ref):
    t = jnp.dot(x_ref[...], w_ref[0], preferred_element_type=jnp.float32)
    col = lax.broadcasted_iota(jnp.int32, (BN, NCOLS), 1)
    out_ref[0] = jnp.where(col < HID, jnp.pad(t, ((0, 0), (0, 16))),
                           jnp.where(col == HID, 1.0, 0.0))


def _table23_body(h_ref, w2_ref, w3_ref, out_ref):
    t2 = jnp.dot(h_ref[...], w2_ref[0], preferred_element_type=jnp.float32)
    t3 = jnp.dot(h_ref[...], w3_ref[0], preferred_element_type=jnp.float32)
    out_ref[0] = jnp.concatenate([t2, t3], axis=1)


def _combine1_body(x_ref, agg_ref, root_ref, b_ref, out_ref, inv_ref):
    acc = jnp.dot(x_ref[...], root_ref[...],
                  preferred_element_type=jnp.float32)
    acc = acc + b_ref[...]
    invs = []
    for r in range(R):
        a = agg_ref[r]
        cnt = a[:, HID:HID + 1]
        inv = 1.0 / jnp.maximum(cnt, 1.0)
        invs.append(inv)
        acc = acc + a[:, :HID] * inv
    out_ref[...] = jnp.maximum(acc, 0.0)
    inv_ref[...] = jnp.concatenate(invs, axis=1)


def _combine23_body(h_ref, agg_ref, inv_ref, root2_ref, b2_ref, root3_ref,
                    b3_ref, eps_ref, mean_ref, logstd_ref, z_ref):
    m = jnp.dot(h_ref[...], root2_ref[...], preferred_element_type=jnp.float32)
    m = m + b2_ref[...]
    g = jnp.dot(h_ref[...], root3_ref[...], preferred_element_type=jnp.float32)
    g = g + b3_ref[...]
    for r in range(R):
        a = agg_ref[r]
        inv = inv_ref[:, r:r + 1]
        m = m + a[:, :OUT] * inv
        g = g + a[:, OUT:2 * OUT] * inv
    mean_ref[...] = m
    logstd_ref[...] = g
    z_ref[...] = m + eps_ref[...] * jnp.exp(g)


BP = 512


def _decoder_body(zs_ref, zd_ref, wdt_ref, wdb_ref, bd_ref,
                  wlt_ref, blt_ref, out_ref):
    share = jnp.dot(zs_ref[...], wdt_ref[...],
                    preferred_element_type=jnp.float32)
    share = share + jnp.dot(zd_ref[...], wdb_ref[...],
                            preferred_element_type=jnp.float32)
    share = jnp.maximum(share + bd_ref[...], 0.0)
    o = jnp.dot(share, wlt_ref[...], preferred_element_type=jnp.float32)
    o = o + blt_ref[...]
    col = lax.broadcasted_iota(jnp.int32, (BP, 16), 1)
    out_ref[...] = jnp.where(col == 0, jax.nn.sigmoid(o), o)


def _table1(x, W1):
    return pl.pallas_call(
        _table1_body,
        grid=(R, N // BN),
        in_specs=[
            pl.BlockSpec((BN, IN), lambda r, i: (i, 0)),
            pl.BlockSpec((1, IN, HID), lambda r, i: (r, 0, 0)),
        ],
        out_specs=pl.BlockSpec((1, BN, NCOLS), lambda r, i: (r, i, 0)),
        out_shape=jax.ShapeDtypeStruct((R, N, NCOLS), jnp.float32),
    )(x, W1)


def _table23(h, W2, W3):
    return pl.pallas_call(
        _table23_body,
        grid=(R, N // BN),
        in_specs=[
            pl.BlockSpec((BN, HID), lambda r, i: (i, 0)),
            pl.BlockSpec((1, HID, OUT), lambda r, i: (r, 0, 0)),
            pl.BlockSpec((1, HID, OUT), lambda r, i: (r, 0, 0)),
        ],
        out_specs=pl.BlockSpec((1, BN, 2 * OUT), lambda r, i: (r, i, 0)),
        out_shape=jax.ShapeDtypeStruct((R, N, 2 * OUT), jnp.float32),
    )(h, W2, W3)


def _combine1(x, agg1, root1, b1):
    return pl.pallas_call(
        _combine1_body,
        grid=(N // BN,),
        in_specs=[
            pl.BlockSpec((BN, IN), lambda i: (i, 0)),
            pl.BlockSpec((R, BN, NCOLS), lambda i: (0, i, 0)),
            pl.BlockSpec((IN, HID), lambda i: (0, 0)),
            pl.BlockSpec((1, HID), lambda i: (0, 0)),
        ],
        out_specs=(pl.BlockSpec((BN, HID), lambda i: (i, 0)),
                   pl.BlockSpec((BN, R), lambda i: (i, 0))),
        out_shape=(jax.ShapeDtypeStruct((N, HID), jnp.float32),
                   jax.ShapeDtypeStruct((N, R), jnp.float32)),
    )(x, agg1, root1, b1)


def _combine23(h, agg2, invt, root2, b2, root3, b3, eps):
    out128 = jax.ShapeDtypeStruct((N, OUT), jnp.float32)
    spec128 = pl.BlockSpec((BN, OUT), lambda i: (i, 0))
    return pl.pallas_call(
        _combine23_body,
        grid=(N // BN,),
        in_specs=[
            pl.BlockSpec((BN, HID), lambda i: (i, 0)),
            pl.BlockSpec((R, BN, 2 * OUT), lambda i: (0, i, 0)),
            pl.BlockSpec((BN, R), lambda i: (i, 0)),
            pl.BlockSpec((HID, OUT), lambda i: (0, 0)),
            pl.BlockSpec((1, OUT), lambda i: (0, 0)),
            pl.BlockSpec((HID, OUT), lambda i: (0, 0)),
            pl.BlockSpec((1, OUT), lambda i: (0, 0)),
            pl.BlockSpec((BN, OUT), lambda i: (i, 0)),
        ],
        out_specs=(spec128, spec128, spec128),
        out_shape=(out128, out128, out128),
    )(h, agg2, invt, root2, b2, root3, b3, eps)


def _decoder(zs, zd, Wdt, Wdb, bd, Wlt, blt):
    return pl.pallas_call(
        _decoder_body,
        grid=(PPAD // BP,),
        in_specs=[
            pl.BlockSpec((BP, OUT), lambda i: (i, 0)),
            pl.BlockSpec((BP, OUT), lambda i: (i, 0)),
            pl.BlockSpec((OUT, 128), lambda i: (0, 0)),
            pl.BlockSpec((OUT, 128), lambda i: (0, 0)),
            pl.BlockSpec((1, 128), lambda i: (0, 0)),
            pl.BlockSpec((128, 16), lambda i: (0, 0)),
            pl.BlockSpec((1, 16), lambda i: (0, 0)),
        ],
        out_specs=pl.BlockSpec((BP, 16), lambda i: (i, 0)),
        out_shape=jax.ShapeDtypeStruct((PPAD, 16), jnp.float32),
    )(zs, zd, Wdt, Wdb, bd, Wlt, blt)


def kernel(x, edge_index, edge_type, pos_edge_index, eps,
           W1, root1, b1, W2, root2, b2, W3, root3, b3,
           Wd, bd, Wl, bl, Wt, bt):
    esrc = edge_index[0]
    edst = edge_index[1]
    etyp = edge_type
    zb = jnp.zeros((16, NCOLS), jnp.float32)
    zb2 = jnp.zeros((16, 2 * OUT), jnp.float32)

    t1 = _table1(x, W1).reshape(R * N, NCOLS)
    agg1 = _seg_agg1(t1, esrc, edst, etyp, zb)
    h, invt = _combine1(x, agg1, root1, b1.reshape(1, HID))
    t23 = _table23(h, W2, W3).reshape(R * N, 2 * OUT)
    agg23 = _seg_agg2(t23, esrc, edst, etyp, zb2)
    mean, logstd, z = _combine23(h, agg23, invt, root2, b2.reshape(1, OUT),
                                 root3, b3.reshape(1, OUT), eps)

    npad = PPAD - P
    psrc = jnp.pad(pos_edge_index[0], (0, npad))
    pdst = jnp.pad(pos_edge_index[1], (0, npad))
    zs, zd = _pair_gather(psrc, pdst, z)

    Wlt = jnp.concatenate([Wl, Wt, jnp.zeros((128, 7), jnp.float32)], axis=1)
    blt = jnp.concatenate([bl, bt, jnp.zeros((7,), jnp.float32)]).reshape(1, 16)
    out16 = _decoder(zs, zd, Wd[:OUT], Wd[OUT:], bd.reshape(1, 128), Wlt, blt)

    link_pred = out16[:P, 0]
    type_pred = out16[:P, 1:1 + R]
    return (link_pred, type_pred, mean, logstd, z)


# speedup vs baseline: 14.9274x; 1.0321x over previous
import functools

import jax
import jax.numpy as jnp
from jax import lax
from jax.experimental import pallas as pl
from jax.experimental.pallas import tpu as pltpu
from jax.experimental.pallas import tpu_sc as plsc

N = 10000
E = 320000
P = 100000
IN, HID, OUT, R = 128, 256, 128, 8

NUM_SC = 2
NUM_TILES = 16
EPT = E // NUM_TILES
HALF_N = N // NUM_SC
KROWS = 32

_mesh = plsc.VectorSubcoreMesh(core_axis_name="c", subcore_axis_name="s")


SB = 2000
NBUF = 4
QPAD = NBUF * KROWS
QCAP = SB + QPAD
NCOLS = 256
NC = 500
NCHUNK = HALF_N // NC


def _seg_agg_body(with_cnt, tab_h, esrc_h, edst_h, etyp_h, zb_h, ones_h,
                  out_h, *rest):
    if with_cnt:
        cnt_h = rest[0]
        rest = rest[1:]
    else:
        cnt_h = None
    (sbuf, dbuf, tbuf, qsrc, qcid, rows, zb, onesv, acc, cacc,
     gsem, ssem) = rest
    c = lax.axis_index("c")
    s = lax.axis_index("s")
    rpt = (R * NC) // NUM_TILES
    ebase = s * EPT
    pltpu.sync_copy(zb_h, zb)
    if with_cnt:
        pltpu.sync_copy(ones_h, onesv)
    node0 = c * HALF_N
    lane = jnp.arange(16, dtype=jnp.int32)
    pad_cid = R * NC + lane
    nz = (rpt + 15) // 16

    def chunk_body(chunk, _c):
        lo = node0 + chunk * NC
        zbase = s * rpt
        for zi in range(nz):
            nrow = min(16, rpt - zi * 16)
            pltpu.sync_copy(zb.at[pl.ds(0, nrow)],
                            acc.at[pl.ds(zbase + zi * 16, nrow)])
            if with_cnt:
                pltpu.sync_copy(zb.at[pl.ds(0, nrow), pl.ds(0, 16)],
                                cacc.at[pl.ds(zbase + zi * 16, nrow)])
        plsc.subcore_barrier()

        for eh, eb in ((esrc_h, sbuf), (edst_h, dbuf), (etyp_h, tbuf)):
            pltpu.async_copy(eh.at[pl.ds(ebase, SB)], eb.at[0], ssem)

        def block_body(b, _b):
            cur = lax.rem(b, 2)
            off = ebase + b * SB
            for eh, eb in ((esrc_h, sbuf), (edst_h, dbuf), (etyp_h, tbuf)):
                pltpu.make_async_copy(eh.at[pl.ds(off, SB)], eb.at[cur],
                                      ssem).wait()

            @pl.when(b + 1 < EPT // SB)
            def _prefetch():
                noff = off + SB
                nxt = lax.rem(b + 1, 2)
                for eh, eb in ((esrc_h, sbuf), (edst_h, dbuf), (etyp_h, tbuf)):
                    pltpu.async_copy(eh.at[pl.ds(noff, SB)], eb.at[nxt], ssem)

            def scan_body(i, qn):
                for u in range(5):
                    q = i * 80 + u * 16
                    sv = sbuf[cur, pl.ds(q, 16)]
                    dv = dbuf[cur, pl.ds(q, 16)]
                    tv = tbuf[cur, pl.ds(q, 16)]
                    m = (dv >= lo) & (dv < lo + NC)
                    mi = m.astype(jnp.int32)
                    pos = qn + plsc.cumsum(mi) - 1
                    gid = tv * N + sv
                    cid = tv * NC + (dv - lo)
                    plsc.store_scatter(qsrc, [pos], gid, mask=m)
                    plsc.store_scatter(qcid, [pos], cid, mask=m)
                    qn = qn + jnp.sum(mi)
                return qn

            qn = lax.fori_loop(0, SB // 80, scan_body, jnp.int32(0))

            for t in range(QPAD // 16):
                qsrc[pl.ds(qn + t * 16, 16)] = lane + 16 * t
                qcid[pl.ds(qn + t * 16, 16)] = pad_cid

            def drain(j4, _):
                qoff = j4 * (NBUF * KROWS)
                cps = [
                    pltpu.async_copy(
                        tab_h.at[qsrc.at[pl.ds(qoff + t * KROWS, KROWS)]],
                        rows.at[t], gsem)
                    for t in range(NBUF)
                ]
                for t in range(NBUF):
                    cps[t].wait()
                    for kk in range(KROWS // 16):
                        cvec = qcid[pl.ds(qoff + t * KROWS + kk * 16, 16)]
                        pltpu.sync_copy(rows.at[t, pl.ds(kk * 16, 16)],
                                        acc.at[cvec], add=True)
                        if with_cnt:
                            pltpu.sync_copy(onesv, cacc.at[cvec], add=True)
                return 0

            nq4 = (qn + QPAD - 1) // QPAD
            lax.fori_loop(0, nq4, drain, 0)
            return 0

        lax.fori_loop(0, EPT // SB, block_body, 0)
        plsc.subcore_barrier()

        rr = (s * rpt) // NC
        roff = (s * rpt) % NC
        pltpu.sync_copy(acc.at[pl.ds(s * rpt, rpt)],
                        out_h.at[rr, pl.ds(lo + roff, rpt)])
        if with_cnt:
            pltpu.sync_copy(cacc.at[pl.ds(s * rpt, rpt)],
                            cnt_h.at[rr, pl.ds(lo + roff, rpt)])
        return 0

    lax.fori_loop(0, NCHUNK, chunk_body, 0)


def _make_seg_agg(with_cnt):
  agg_t = jax.ShapeDtypeStruct((R, N, NCOLS), jnp.float32)
  cnt_t = jax.ShapeDtypeStruct((R, N, 16), jnp.float32)
  return pl.kernel(
    functools.partial(_seg_agg_body, with_cnt),
    out_type=(agg_t, cnt_t) if with_cnt else agg_t,
    mesh=_mesh,
    scratch_types=[
        pltpu.VMEM((2, SB), jnp.int32),
        pltpu.VMEM((2, SB), jnp.int32),
        pltpu.VMEM((2, SB), jnp.int32),
        pltpu.VMEM((QCAP,), jnp.int32),
        pltpu.VMEM((QCAP,), jnp.int32),
        pltpu.VMEM((NBUF, KROWS, NCOLS), jnp.float32),
        pltpu.VMEM((16, NCOLS), jnp.float32),
        pltpu.VMEM((16, 16), jnp.float32),
        pltpu.VMEM_SHARED((R * NC + 16, NCOLS), jnp.float32),
        pltpu.VMEM_SHARED((R * NC + 16, 16), jnp.float32),
        pltpu.SemaphoreType.DMA,
        pltpu.SemaphoreType.DMA,
    ],
    compiler_params=pltpu.CompilerParams(
        use_tc_tiling_on_sc=False, needs_layout_passes=False),
  )


_seg_agg1 = _make_seg_agg(True)
_seg_agg2 = _make_seg_agg(False)


PPAD = 100352
PPW = PPAD // (NUM_SC * NUM_TILES)
KP = 32


def _pair_gather_body(psrc_h, pdst_h, z_h, zs_h, zd_h,
                      sidx, didx, zsb, zdb, sem1, sem2):
    c = lax.axis_index("c")
    s = lax.axis_index("s")
    wid = s * NUM_SC + c
    base = wid * PPW
    pltpu.sync_copy(psrc_h.at[pl.ds(base, PPW)], sidx)
    pltpu.sync_copy(pdst_h.at[pl.ds(base, PPW)], didx)

    def loop(j, _):
        cps = []
        for t in range(2):
            q = j * 2 * KP + t * KP
            cps.append((
                pltpu.async_copy(z_h.at[sidx.at[pl.ds(q, KP)]],
                                 zsb.at[t], sem1),
                pltpu.async_copy(z_h.at[didx.at[pl.ds(q, KP)]],
                                 zdb.at[t], sem2),
            ))
        for t in range(2):
            g1, g2 = cps[t]
            g1.wait()
            g2.wait()
            q = base + j * 2 * KP + t * KP
            pltpu.sync_copy(zsb.at[t], zs_h.at[pl.ds(q, KP)])
            pltpu.sync_copy(zdb.at[t], zd_h.at[pl.ds(q, KP)])
        return 0

    lax.fori_loop(0, PPW // (2 * KP), loop, 0)


_pair_gather = pl.kernel(
    _pair_gather_body,
    out_type=(jax.ShapeDtypeStruct((PPAD, OUT), jnp.float32),
              jax.ShapeDtypeStruct((PPAD, OUT), jnp.float32)),
    mesh=_mesh,
    scratch_types=[
        pltpu.VMEM((PPW,), jnp.int32),
        pltpu.VMEM((PPW,), jnp.int32),
        pltpu.VMEM((2, KP, OUT), jnp.float32),
        pltpu.VMEM((2, KP, OUT), jnp.float32),
        pltpu.SemaphoreType.DMA,
        pltpu.SemaphoreType.DMA,
    ],
    compiler_params=pltpu.CompilerParams(
        use_tc_tiling_on_sc=False, needs_layout_passes=False),
)


BN = 400


def _table1_body(x_ref, w_ref, out_ref):
    out_ref[0] = jnp.dot(x_ref[...], w_ref[0],
                         preferred_element_type=jnp.float32)


def _table23_body(h_ref, w2_ref, w3_ref, out_ref):
    t2 = jnp.dot(h_ref[...], w2_ref[0], preferred_element_type=jnp.float32)
    t3 = jnp.dot(h_ref[...], w3_ref[0], preferred_element_type=jnp.float32)
    out_ref[0] = jnp.concatenate([t2, t3], axis=1)


def _combine1_body(x_ref, agg_ref, cnt_ref, root_ref, b_ref, out_ref,
                   inv_ref):
    acc = jnp.dot(x_ref[...], root_ref[...],
                  preferred_element_type=jnp.float32)
    acc = acc + b_ref[...]
    invs = []
    for r in range(R):
        a = agg_ref[r]
        cnt = cnt_ref[r][:, 0:1]
        inv = 1.0 / jnp.maximum(cnt, 1.0)
        invs.append(inv)
        acc = acc + a * inv
    out_ref[...] = jnp.maximum(acc, 0.0)
    inv_ref[...] = jnp.concatenate(invs, axis=1)


def _combine23_body(h_ref, agg_ref, inv_ref, root2_ref, b2_ref, root3_ref,
                    b3_ref, eps_ref, mean_ref, logstd_ref, z_ref):
    m = jnp.dot(h_ref[...], root2_ref[...], preferred_element_type=jnp.float32)
    m = m + b2_ref[...]
    g = jnp.dot(h_ref[...], root3_ref[...], preferred_element_type=jnp.float32)
    g = g + b3_ref[...]
    for r in range(R):
        a = agg_ref[r]
        inv = inv_ref[:, r:r + 1]
        m = m + a[:, :OUT] * inv
        g = g + a[:, OUT:2 * OUT] * inv
    mean_ref[...] = m
    logstd_ref[...] = g
    z_ref[...] = m + eps_ref[...] * jnp.exp(g)


BP = 512


def _decoder_body(zs_ref, zd_ref, wdt_ref, wdb_ref, bd_ref,
                  wlt_ref, blt_ref, out_ref):
    share = jnp.dot(zs_ref[...], wdt_ref[...],
                    preferred_element_type=jnp.float32)
    share = share + jnp.dot(zd_ref[...], wdb_ref[...],
                            preferred_element_type=jnp.float32)
    share = jnp.maximum(share + bd_ref[...], 0.0)
    o = jnp.dot(share, wlt_ref[...], preferred_element_type=jnp.float32)
    o = o + blt_ref[...]
    col = lax.broadcasted_iota(jnp.int32, (BP, 16), 1)
    out_ref[...] = jnp.where(col == 0, jax.nn.sigmoid(o), o)


def _table1(x, W1):
    return pl.pallas_call(
        _table1_body,
        grid=(R, N // BN),
        in_specs=[
            pl.BlockSpec((BN, IN), lambda r, i: (i, 0)),
            pl.BlockSpec((1, IN, HID), lambda r, i: (r, 0, 0)),
        ],
        out_specs=pl.BlockSpec((1, BN, NCOLS), lambda r, i: (r, i, 0)),
        out_shape=jax.ShapeDtypeStruct((R, N, NCOLS), jnp.float32),
    )(x, W1)


def _table23(h, W2, W3):
    return pl.pallas_call(
        _table23_body,
        grid=(R, N // BN),
        in_specs=[
            pl.BlockSpec((BN, HID), lambda r, i: (i, 0)),
            pl.BlockSpec((1, HID, OUT), lambda r, i: (r, 0, 0)),
            pl.BlockSpec((1, HID, OUT), lambda r, i: (r, 0, 0)),
        ],
        out_specs=pl.BlockSpec((1, BN, 2 * OUT), lambda r, i: (r, i, 0)),
        out_shape=jax.ShapeDtypeStruct((R, N, 2 * OUT), jnp.float32),
    )(h, W2, W3)


def _combine1(x, agg1, cnt1, root1, b1):
    return pl.pallas_call(
        _combine1_body,
        grid=(N // BN,),
        in_specs=[
            pl.BlockSpec((BN, IN), lambda i: (i, 0)),
            pl.BlockSpec((R, BN, NCOLS), lambda i: (0, i, 0)),
            pl.BlockSpec((R, BN, 16), lambda i: (0, i, 0)),
            pl.BlockSpec((IN, HID), lambda i: (0, 0)),
            pl.BlockSpec((1, HID), lambda i: (0, 0)),
        ],
        out_specs=(pl.BlockSpec((BN, HID), lambda i: (i, 0)),
                   pl.BlockSpec((BN, R), lambda i: (i, 0))),
        out_shape=(jax.ShapeDtypeStruct((N, HID), jnp.float32),
                   jax.ShapeDtypeStruct((N, R), jnp.float32)),
    )(x, agg1, cnt1, root1, b1)


def _combine23(h, agg2, invt, root2, b2, root3, b3, eps):
    out128 = jax.ShapeDtypeStruct((N, OUT), jnp.float32)
    spec128 = pl.BlockSpec((BN, OUT), lambda i: (i, 0))
    return pl.pallas_call(
        _combine23_body,
        grid=(N // BN,),
        in_specs=[
            pl.BlockSpec((BN, HID), lambda i: (i, 0)),
            pl.BlockSpec((R, BN, 2 * OUT), lambda i: (0, i, 0)),
            pl.BlockSpec((BN, R), lambda i: (i, 0)),
            pl.BlockSpec((HID, OUT), lambda i: (0, 0)),
            pl.BlockSpec((1, OUT), lambda i: (0, 0)),
            pl.BlockSpec((HID, OUT), lambda i: (0, 0)),
            pl.BlockSpec((1, OUT), lambda i: (0, 0)),
            pl.BlockSpec((BN, OUT), lambda i: (i, 0)),
        ],
        out_specs=(spec128, spec128, spec128),
        out_shape=(out128, out128, out128),
    )(h, agg2, invt, root2, b2, root3, b3, eps)


def _decoder(zs, zd, Wdt, Wdb, bd, Wlt, blt):
    return pl.pallas_call(
        _decoder_body,
        grid=(PPAD // BP,),
        in_specs=[
            pl.BlockSpec((BP, OUT), lambda i: (i, 0)),
            pl.BlockSpec((BP, OUT), lambda i: (i, 0)),
            pl.BlockSpec((OUT, 128), lambda i: (0, 0)),
            pl.BlockSpec((OUT, 128), lambda i: (0, 0)),
            pl.BlockSpec((1, 128), lambda i: (0, 0)),
            pl.BlockSpec((128, 16), lambda i: (0, 0)),
            pl.BlockSpec((1, 16), lambda i: (0, 0)),
        ],
        out_specs=pl.BlockSpec((BP, 16), lambda i: (i, 0)),
        out_shape=jax.ShapeDtypeStruct((PPAD, 16), jnp.float32),
    )(zs, zd, Wdt, Wdb, bd, Wlt, blt)


def kernel(x, edge_index, edge_type, pos_edge_index, eps,
           W1, root1, b1, W2, root2, b2, W3, root3, b3,
           Wd, bd, Wl, bl, Wt, bt):
    esrc = edge_index[0]
    edst = edge_index[1]
    etyp = edge_type
    zb = jnp.zeros((16, NCOLS), jnp.float32)
    ones16 = jnp.ones((16, 16), jnp.float32)

    t1 = _table1(x, W1).reshape(R * N, NCOLS)
    agg1, cnt1 = _seg_agg1(t1, esrc, edst, etyp, zb, ones16)
    h, invt = _combine1(x, agg1, cnt1, root1, b1.reshape(1, HID))
    t23 = _table23(h, W2, W3).reshape(R * N, 2 * OUT)
    agg23 = _seg_agg2(t23, esrc, edst, etyp, zb, ones16)
    mean, logstd, z = _combine23(h, agg23, invt, root2, b2.reshape(1, OUT),
                                 root3, b3.reshape(1, OUT), eps)

    npad = PPAD - P
    psrc = jnp.pad(pos_edge_index[0], (0, npad))
    pdst = jnp.pad(pos_edge_index[1], (0, npad))
    zs, zd = _pair_gather(psrc, pdst, z)

    Wlt = jnp.concatenate([Wl, Wt, jnp.zeros((128, 7), jnp.float32)], axis=1)
    blt = jnp.concatenate([bl, bt, jnp.zeros((7,), jnp.float32)]).reshape(1, 16)
    out16 = _decoder(zs, zd, Wd[:OUT], Wd[OUT:], bd.reshape(1, 128), Wlt, blt)

    link_pred = out16[:P, 0]
    type_pred = out16[:P, 1:1 + R]
    return (link_pred, type_pred, mean, logstd, z)


# speedup vs baseline: 16.2922x; 1.0914x over previous
import functools

import jax
import jax.numpy as jnp
from jax import lax
from jax.experimental import pallas as pl
from jax.experimental.pallas import tpu as pltpu
from jax.experimental.pallas import tpu_sc as plsc

N = 10000
E = 320000
P = 100000
IN, HID, OUT, R = 128, 256, 128, 8

NUM_SC = 2
NUM_TILES = 16
EPT = E // NUM_TILES
HALF_N = N // NUM_SC
KROWS = 32

_mesh = plsc.VectorSubcoreMesh(core_axis_name="c", subcore_axis_name="s")


SB = 2000
NBUF = 4
QPAD = NBUF * KROWS
QCAP = SB + QPAD
NCOLS = 256
NC = 500
NCHUNK = HALF_N // NC


def _seg_agg_body(with_cnt, tab_h, esrc_h, edst_h, etyp_h, zb_h, ones_h,
                  out_h, *rest):
    if with_cnt:
        cnt_h = rest[0]
        rest = rest[1:]
    else:
        cnt_h = None
    (sbuf, dbuf, tbuf, qsrc, qcid, rows, zb, onesv, acc, cacc,
     gsem, ssem) = rest
    c = lax.axis_index("c")
    s = lax.axis_index("s")
    rpt = (R * NC) // NUM_TILES
    ebase = s * EPT
    pltpu.sync_copy(zb_h, zb)
    if with_cnt:
        pltpu.sync_copy(ones_h, onesv)
    node0 = c * HALF_N
    lane = jnp.arange(16, dtype=jnp.int32)
    pad_cid = R * NC + lane
    nz = (rpt + 15) // 16

    def chunk_body(chunk, _c):
        lo = node0 + chunk * NC
        zbase = s * rpt
        for zi in range(nz):
            nrow = min(16, rpt - zi * 16)
            pltpu.sync_copy(zb.at[pl.ds(0, nrow)],
                            acc.at[pl.ds(zbase + zi * 16, nrow)])
            if with_cnt:
                pltpu.sync_copy(zb.at[pl.ds(0, nrow), pl.ds(0, 16)],
                                cacc.at[pl.ds(zbase + zi * 16, nrow)])
        plsc.subcore_barrier()

        for eh, eb in ((esrc_h, sbuf), (edst_h, dbuf), (etyp_h, tbuf)):
            pltpu.async_copy(eh.at[pl.ds(ebase, SB)], eb.at[0], ssem)

        def block_body(b, _b):
            cur = lax.rem(b, 2)
            off = ebase + b * SB
            for eh, eb in ((esrc_h, sbuf), (edst_h, dbuf), (etyp_h, tbuf)):
                pltpu.make_async_copy(eh.at[pl.ds(off, SB)], eb.at[cur],
                                      ssem).wait()

            @pl.when(b + 1 < EPT // SB)
            def _prefetch():
                noff = off + SB
                nxt = lax.rem(b + 1, 2)
                for eh, eb in ((esrc_h, sbuf), (edst_h, dbuf), (etyp_h, tbuf)):
                    pltpu.async_copy(eh.at[pl.ds(noff, SB)], eb.at[nxt], ssem)

            def scan_body(i, qn):
                for u in range(5):
                    q = i * 80 + u * 16
                    sv = sbuf[cur, pl.ds(q, 16)]
                    dv = dbuf[cur, pl.ds(q, 16)]
                    tv = tbuf[cur, pl.ds(q, 16)]
                    m = (dv >= lo) & (dv < lo + NC)
                    mi = m.astype(jnp.int32)
                    pos = qn + plsc.cumsum(mi) - 1
                    gid = tv * N + sv
                    cid = tv * NC + (dv - lo)
                    plsc.store_scatter(qsrc, [pos], gid, mask=m)
                    plsc.store_scatter(qcid, [pos], cid, mask=m)
                    qn = qn + jnp.sum(mi)
                return qn

            qn = lax.fori_loop(0, SB // 80, scan_body, jnp.int32(0))

            for t in range(QPAD // 16):
                qsrc[pl.ds(qn + t * 16, 16)] = lane + 16 * t
                qcid[pl.ds(qn + t * 16, 16)] = pad_cid

            def drain(j4, _):
                qoff = j4 * (NBUF * KROWS)
                cps = [
                    pltpu.async_copy(
                        tab_h.at[qsrc.at[pl.ds(qoff + t * KROWS, KROWS)]],
                        rows.at[t], gsem)
                    for t in range(NBUF)
                ]
                for t in range(NBUF):
                    cps[t].wait()
                    for kk in range(KROWS // 16):
                        cvec = qcid[pl.ds(qoff + t * KROWS + kk * 16, 16)]
                        pltpu.sync_copy(rows.at[t, pl.ds(kk * 16, 16)],
                                        acc.at[cvec], add=True)
                        if with_cnt:
                            pltpu.sync_copy(onesv, cacc.at[cvec], add=True)
                return 0

            nq4 = (qn + QPAD - 1) // QPAD
            lax.fori_loop(0, nq4, drain, 0)
            return 0

        lax.fori_loop(0, EPT // SB, block_body, 0)
        plsc.subcore_barrier()

        rr = (s * rpt) // NC
        roff = (s * rpt) % NC
        pltpu.sync_copy(acc.at[pl.ds(s * rpt, rpt)],
                        out_h.at[rr, pl.ds(lo + roff, rpt)])
        if with_cnt:
            pltpu.sync_copy(cacc.at[pl.ds(s * rpt, rpt)],
                            cnt_h.at[rr, pl.ds(lo + roff, rpt)])
        return 0

    lax.fori_loop(0, NCHUNK, chunk_body, 0)


def _make_seg_agg(with_cnt):
  agg_t = jax.ShapeDtypeStruct((R, N, NCOLS), jnp.float32)
  cnt_t = jax.ShapeDtypeStruct((R, N, 16), jnp.float32)
  return pl.kernel(
    functools.partial(_seg_agg_body, with_cnt),
    out_type=(agg_t, cnt_t) if with_cnt else agg_t,
    mesh=_mesh,
    scratch_types=[
        pltpu.VMEM((2, SB), jnp.int32),
        pltpu.VMEM((2, SB), jnp.int32),
        pltpu.VMEM((2, SB), jnp.int32),
        pltpu.VMEM((QCAP,), jnp.int32),
        pltpu.VMEM((QCAP,), jnp.int32),
        pltpu.VMEM((NBUF, KROWS, NCOLS), jnp.float32),
        pltpu.VMEM((16, NCOLS), jnp.float32),
        pltpu.VMEM((16, 16), jnp.float32),
        pltpu.VMEM_SHARED((R * NC + 16, NCOLS), jnp.float32),
        pltpu.VMEM_SHARED((R * NC + 16, 16), jnp.float32),
        pltpu.SemaphoreType.DMA,
        pltpu.SemaphoreType.DMA,
    ],
    compiler_params=pltpu.CompilerParams(
        use_tc_tiling_on_sc=False, needs_layout_passes=False),
  )


_seg_agg1 = _make_seg_agg(True)
_seg_agg2 = _make_seg_agg(False)


PPAD = 100352
PPW = PPAD // (NUM_SC * NUM_TILES)
KP = 32


def _pair_gather_body(psrc_h, pdst_h, z_h, zs_h, zd_h,
                      sidx, didx, zsb, zdb, sem1, sem2):
    c = lax.axis_index("c")
    s = lax.axis_index("s")
    wid = s * NUM_SC + c
    base = wid * PPW
    pltpu.sync_copy(psrc_h.at[pl.ds(base, PPW)], sidx)
    pltpu.sync_copy(pdst_h.at[pl.ds(base, PPW)], didx)

    def loop(j, _):
        cps = []
        for t in range(2):
            q = j * 2 * KP + t * KP
            cps.append((
                pltpu.async_copy(z_h.at[sidx.at[pl.ds(q, KP)]],
                                 zsb.at[t], sem1),
                pltpu.async_copy(z_h.at[didx.at[pl.ds(q, KP)]],
                                 zdb.at[t], sem2),
            ))
        for t in range(2):
            g1, g2 = cps[t]
            g1.wait()
            g2.wait()
            q = base + j * 2 * KP + t * KP
            pltpu.sync_copy(zsb.at[t], zs_h.at[pl.ds(q, KP)])
            pltpu.sync_copy(zdb.at[t], zd_h.at[pl.ds(q, KP)])
        return 0

    lax.fori_loop(0, PPW // (2 * KP), loop, 0)


_pair_gather = pl.kernel(
    _pair_gather_body,
    out_type=(jax.ShapeDtypeStruct((PPAD, OUT), jnp.float32),
              jax.ShapeDtypeStruct((PPAD, OUT), jnp.float32)),
    mesh=_mesh,
    scratch_types=[
        pltpu.VMEM((PPW,), jnp.int32),
        pltpu.VMEM((PPW,), jnp.int32),
        pltpu.VMEM((2, KP, OUT), jnp.float32),
        pltpu.VMEM((2, KP, OUT), jnp.float32),
        pltpu.SemaphoreType.DMA,
        pltpu.SemaphoreType.DMA,
    ],
    compiler_params=pltpu.CompilerParams(
        use_tc_tiling_on_sc=False, needs_layout_passes=False),
)


BN = 1000


def _table1_body(x_ref, w_ref, out_ref):
    out_ref[0] = jnp.dot(x_ref[...], w_ref[0],
                         preferred_element_type=jnp.float32)


def _table23_body(h_ref, w2_ref, w3_ref, out_ref):
    t2 = jnp.dot(h_ref[...], w2_ref[0], preferred_element_type=jnp.float32)
    t3 = jnp.dot(h_ref[...], w3_ref[0], preferred_element_type=jnp.float32)
    out_ref[0] = jnp.concatenate([t2, t3], axis=1)


def _combine1_body(x_ref, agg_ref, cnt_ref, root_ref, b_ref, out_ref,
                   inv_ref):
    acc = jnp.dot(x_ref[...], root_ref[...],
                  preferred_element_type=jnp.float32)
    acc = acc + b_ref[...]
    invs = []
    for r in range(R):
        a = agg_ref[r]
        cnt = cnt_ref[r][:, 0:1]
        inv = 1.0 / jnp.maximum(cnt, 1.0)
        invs.append(inv)
        acc = acc + a * inv
    out_ref[...] = jnp.maximum(acc, 0.0)
    inv_ref[...] = jnp.concatenate(invs, axis=1)


def _combine23_body(h_ref, agg_ref, inv_ref, root2_ref, b2_ref, root3_ref,
                    b3_ref, eps_ref, mean_ref, logstd_ref, z_ref):
    m = jnp.dot(h_ref[...], root2_ref[...], preferred_element_type=jnp.float32)
    m = m + b2_ref[...]
    g = jnp.dot(h_ref[...], root3_ref[...], preferred_element_type=jnp.float32)
    g = g + b3_ref[...]
    for r in range(R):
        a = agg_ref[r]
        inv = inv_ref[:, r:r + 1]
        m = m + a[:, :OUT] * inv
        g = g + a[:, OUT:2 * OUT] * inv
    mean_ref[...] = m
    logstd_ref[...] = g
    z_ref[...] = m + eps_ref[...] * jnp.exp(g)


BP = 1024


def _decoder_body(zs_ref, zd_ref, wdt_ref, wdb_ref, bd_ref,
                  wlt_ref, blt_ref, out_ref):
    share = jnp.dot(zs_ref[...], wdt_ref[...],
                    preferred_element_type=jnp.float32)
    share = share + jnp.dot(zd_ref[...], wdb_ref[...],
                            preferred_element_type=jnp.float32)
    share = jnp.maximum(share + bd_ref[...], 0.0)
    o = jnp.dot(share, wlt_ref[...], preferred_element_type=jnp.float32)
    o = o + blt_ref[...]
    col = lax.broadcasted_iota(jnp.int32, (BP, 16), 1)
    out_ref[...] = jnp.where(col == 0, jax.nn.sigmoid(o), o)


def _table1(x, W1):
    return pl.pallas_call(
        _table1_body,
        grid=(R, N // BN),
        in_specs=[
            pl.BlockSpec((BN, IN), lambda r, i: (i, 0)),
            pl.BlockSpec((1, IN, HID), lambda r, i: (r, 0, 0)),
        ],
        out_specs=pl.BlockSpec((1, BN, NCOLS), lambda r, i: (r, i, 0)),
        out_shape=jax.ShapeDtypeStruct((R, N, NCOLS), jnp.float32),
    )(x, W1)


def _table23(h, W2, W3):
    return pl.pallas_call(
        _table23_body,
        grid=(R, N // BN),
        in_specs=[
            pl.BlockSpec((BN, HID), lambda r, i: (i, 0)),
            pl.BlockSpec((1, HID, OUT), lambda r, i: (r, 0, 0)),
            pl.BlockSpec((1, HID, OUT), lambda r, i: (r, 0, 0)),
        ],
        out_specs=pl.BlockSpec((1, BN, 2 * OUT), lambda r, i: (r, i, 0)),
        out_shape=jax.ShapeDtypeStruct((R, N, 2 * OUT), jnp.float32),
    )(h, W2, W3)


def _combine1(x, agg1, cnt1, root1, b1):
    return pl.pallas_call(
        _combine1_body,
        grid=(N // BN,),
        in_specs=[
            pl.BlockSpec((BN, IN), lambda i: (i, 0)),
            pl.BlockSpec((R, BN, NCOLS), lambda i: (0, i, 0)),
            pl.BlockSpec((R, BN, 16), lambda i: (0, i, 0)),
            pl.BlockSpec((IN, HID), lambda i: (0, 0)),
            pl.BlockSpec((1, HID), lambda i: (0, 0)),
        ],
        out_specs=(pl.BlockSpec((BN, HID), lambda i: (i, 0)),
                   pl.BlockSpec((BN, R), lambda i: (i, 0))),
        out_shape=(jax.ShapeDtypeStruct((N, HID), jnp.float32),
                   jax.ShapeDtypeStruct((N, R), jnp.float32)),
    )(x, agg1, cnt1, root1, b1)


def _combine23(h, agg2, invt, root2, b2, root3, b3, eps):
    out128 = jax.ShapeDtypeStruct((N, OUT), jnp.float32)
    spec128 = pl.BlockSpec((BN, OUT), lambda i: (i, 0))
    return pl.pallas_call(
        _combine23_body,
        grid=(N // BN,),
        in_specs=[
            pl.BlockSpec((BN, HID), lambda i: (i, 0)),
            pl.BlockSpec((R, BN, 2 * OUT), lambda i: (0, i, 0)),
            pl.BlockSpec((BN, R), lambda i: (i, 0)),
            pl.BlockSpec((HID, OUT), lambda i: (0, 0)),
            pl.BlockSpec((1, OUT), lambda i: (0, 0)),
            pl.BlockSpec((HID, OUT), lambda i: (0, 0)),
            pl.BlockSpec((1, OUT), lambda i: (0, 0)),
            pl.BlockSpec((BN, OUT), lambda i: (i, 0)),
        ],
        out_specs=(spec128, spec128, spec128),
        out_shape=(out128, out128, out128),
    )(h, agg2, invt, root2, b2, root3, b3, eps)


def _decoder(zs, zd, Wdt, Wdb, bd, Wlt, blt):
    return pl.pallas_call(
        _decoder_body,
        grid=(PPAD // BP,),
        in_specs=[
            pl.BlockSpec((BP, OUT), lambda i: (i, 0)),
            pl.BlockSpec((BP, OUT), lambda i: (i, 0)),
            pl.BlockSpec((OUT, 128), lambda i: (0, 0)),
            pl.BlockSpec((OUT, 128), lambda i: (0, 0)),
            pl.BlockSpec((1, 128), lambda i: (0, 0)),
            pl.BlockSpec((128, 16), lambda i: (0, 0)),
            pl.BlockSpec((1, 16), lambda i: (0, 0)),
        ],
        out_specs=pl.BlockSpec((BP, 16), lambda i: (i, 0)),
        out_shape=jax.ShapeDtypeStruct((PPAD, 16), jnp.float32),
    )(zs, zd, Wdt, Wdb, bd, Wlt, blt)


def kernel(x, edge_index, edge_type, pos_edge_index, eps,
           W1, root1, b1, W2, root2, b2, W3, root3, b3,
           Wd, bd, Wl, bl, Wt, bt):
    esrc = edge_index[0]
    edst = edge_index[1]
    etyp = edge_type
    zb = jnp.zeros((16, NCOLS), jnp.float32)
    ones16 = jnp.ones((16, 16), jnp.float32)

    t1 = _table1(x, W1).reshape(R * N, NCOLS)
    agg1, cnt1 = _seg_agg1(t1, esrc, edst, etyp, zb, ones16)
    h, invt = _combine1(x, agg1, cnt1, root1, b1.reshape(1, HID))
    t23 = _table23(h, W2, W3).reshape(R * N, 2 * OUT)
    agg23 = _seg_agg2(t23, esrc, edst, etyp, zb, ones16)
    mean, logstd, z = _combine23(h, agg23, invt, root2, b2.reshape(1, OUT),
                                 root3, b3.reshape(1, OUT), eps)

    npad = PPAD - P
    psrc = jnp.pad(pos_edge_index[0], (0, npad))
    pdst = jnp.pad(pos_edge_index[1], (0, npad))
    zs, zd = _pair_gather(psrc, pdst, z)

    Wlt = jnp.concatenate([Wl, Wt, jnp.zeros((128, 7), jnp.float32)], axis=1)
    blt = jnp.concatenate([bl, bt, jnp.zeros((7,), jnp.float32)]).reshape(1, 16)
    out16 = _decoder(zs, zd, Wd[:OUT], Wd[OUT:], bd.reshape(1, 128), Wlt, blt)

    link_pred = out16[:P, 0]
    type_pred = out16[:P, 1:1 + R]
    return (link_pred, type_pred, mean, logstd, z)
